# Initial kernel scaffold; baseline (speedup 1.0000x reference)
import jax
import jax.numpy as jnp
from jax.experimental import pallas as pl


def kernel(gender, age, major, grade, gender_tab, age_tab, major_tab, grade_tab, W1, b1, W2, b2):
    raise NotImplementedError("write your pallas kernel here")


# speedup vs baseline: 3.6929x; 3.6929x over previous
import functools

import jax
import jax.numpy as jnp
from jax import lax
from jax.experimental import pallas as pl
from jax.experimental.pallas import tpu as pltpu
from jax.experimental.pallas import tpu_sc as plsc

B = 16384
EMBED = 64
HID = 256
NC, NS = 2, 16
NW = NC * NS
ROWS_PER_W = B // NW
BLK = 128
NBLK = ROWS_PER_W // BLK
LANES = 16
NCHUNK = HID // LANES


def _tc_project(g_ref, a_ref, m_ref, r_ref, w1_ref, b1_ref, b2_ref,
                pgr_o, pa_o, pm_o, b2v_o):
    w1 = w1_ref[...]
    dn = (((1,), (1,)), ((), ()))
    pg = lax.dot_general(g_ref[...], w1[:, 0:64], dn,
                         preferred_element_type=jnp.float32)
    pa = lax.dot_general(a_ref[...], w1[:, 64:128], dn,
                         preferred_element_type=jnp.float32)
    pm = lax.dot_general(m_ref[...], w1[:, 128:192], dn,
                         preferred_element_type=jnp.float32)
    pr = lax.dot_general(r_ref[...], w1[:, 192:256], dn,
                         preferred_element_type=jnp.float32)
    b1 = b1_ref[...]
    pgr = (pg[:, None, :] + pr[None, :, :]).reshape(16, HID) + b1
    pgr_o[...] = pgr
    pa_o[...] = pa
    pm_o[...] = pm
    lane = lax.broadcasted_iota(jnp.int32, (1, LANES), 1)
    b2v_o[...] = jnp.where(lane == 0, 1.0, 0.0) * b2_ref[...]


def _sc_forward(pgr_hbm, pa_hbm, pm_hbm, w2_hbm, b2v_hbm,
                g_hbm, a_hbm, r_hbm, m_hbm, out_hbm,
                pgr_v, pa_v, pm_buf, w2_v, b2_v,
                g_v, a_v, r_v, maj_v, out_v, part_v, sem0, sem1):
    wid = lax.axis_index("s") * NC + lax.axis_index("c")
    base = wid * ROWS_PER_W

    pltpu.sync_copy(pgr_hbm, pgr_v)
    pltpu.sync_copy(pa_hbm, pa_v)
    pltpu.sync_copy(w2_hbm, w2_v)
    pltpu.sync_copy(b2v_hbm, b2_v)
    pltpu.sync_copy(g_hbm.at[wid], g_v.at[pl.ds(0, ROWS_PER_W)])
    pltpu.sync_copy(a_hbm.at[wid], a_v.at[pl.ds(0, ROWS_PER_W)])
    pltpu.sync_copy(r_hbm.at[wid], r_v.at[pl.ds(0, ROWS_PER_W)])
    pltpu.sync_copy(m_hbm.at[wid], maj_v)

    sems = [sem0, sem1]
    descs = [None, None]
    descs[0] = pltpu.async_copy(pm_hbm.at[maj_v.at[0]], pm_buf.at[0], sems[0])

    w2c = [w2_v[pl.ds(LANES * j, LANES)] for j in range(NCHUNK)]
    b2acc = b2_v[...]

    for blk in range(NBLK):
        cur = blk % 2
        if blk + 1 < NBLK:
            nxt = (blk + 1) % 2
            descs[nxt] = pltpu.async_copy(
                pm_hbm.at[maj_v.at[blk + 1]], pm_buf.at[nxt], sems[nxt])
        descs[cur].wait()
        rows = pm_buf.at[cur]

        @pl.loop(0, BLK)
        def _row(r):
            rb = blk * BLK + r
            off_gr = (g_v[pl.ds(rb, LANES)][0] * 8
                      + r_v[pl.ds(rb, LANES)][0]) * HID
            off_a = a_v[pl.ds(rb, LANES)][0] * HID
            acc = b2acc
            for j in range(NCHUNK):
                v = rows[r, pl.ds(LANES * j, LANES)]
                v = v + pa_v[pl.ds(off_a + LANES * j, LANES)]
                v = v + pgr_v[pl.ds(off_gr + LANES * j, LANES)]
                h = jnp.maximum(v, 0.0)
                acc = acc + h * w2c[j]
            part_v[pl.ds(r * LANES, LANES)] = acc

        iota16 = lax.iota(jnp.int32, LANES) * LANES

        @pl.loop(0, BLK // LANES)
        def _red(g16):
            gbase = g16 * (LANES * LANES)
            tot = plsc.load_gather(part_v, [iota16 + gbase])
            for l in range(1, LANES):
                tot = tot + plsc.load_gather(part_v, [iota16 + (gbase + l)])
            out_v[pl.ds(blk * BLK + g16 * LANES, LANES)] = tot

    pltpu.sync_copy(out_v, out_hbm.at[pl.ds(base, ROWS_PER_W)])


def kernel(gender, age, major, grade, gender_tab, age_tab, major_tab,
           grade_tab, W1, b1, W2, b2):
    pgr, pa, pm, b2v = pl.pallas_call(
        _tc_project,
        out_shape=[
            jax.ShapeDtypeStruct((16, HID), jnp.float32),
            jax.ShapeDtypeStruct((100, HID), jnp.float32),
            jax.ShapeDtypeStruct((1000, HID), jnp.float32),
            jax.ShapeDtypeStruct((1, LANES), jnp.float32),
        ],
    )(gender_tab, age_tab, major_tab, grade_tab, W1,
      b1.reshape(1, HID), b2.reshape(1, 1))

    mesh = plsc.VectorSubcoreMesh(core_axis_name="c", subcore_axis_name="s",
                                  num_cores=NC, num_subcores=NS)
    sc = pl.kernel(
        _sc_forward,
        out_type=jax.ShapeDtypeStruct((B,), jnp.float32),
        mesh=mesh,
        compiler_params=pltpu.CompilerParams(needs_layout_passes=False),
        scratch_types=[
            pltpu.VMEM((16 * HID,), jnp.float32),
            pltpu.VMEM((100 * HID,), jnp.float32),
            pltpu.VMEM((2, BLK, HID), jnp.float32),
            pltpu.VMEM((HID,), jnp.float32),
            pltpu.VMEM((LANES,), jnp.float32),
            pltpu.VMEM((ROWS_PER_W + LANES,), jnp.int32),
            pltpu.VMEM((ROWS_PER_W + LANES,), jnp.int32),
            pltpu.VMEM((ROWS_PER_W + LANES,), jnp.int32),
            pltpu.VMEM((NBLK, BLK), jnp.int32),
            pltpu.VMEM((ROWS_PER_W,), jnp.float32),
            pltpu.VMEM((BLK * LANES,), jnp.float32),
            pltpu.SemaphoreType.DMA,
            pltpu.SemaphoreType.DMA,
        ],
    )
    i32 = jnp.int32
    return sc(pgr.reshape(-1), pa.reshape(-1), pm, W2.reshape(-1),
              b2v.reshape(-1),
              gender.astype(i32).reshape(NW, ROWS_PER_W),
              age.astype(i32).reshape(NW, ROWS_PER_W),
              grade.astype(i32).reshape(NW, ROWS_PER_W),
              major.astype(i32).reshape(NW, NBLK, BLK))


# speedup vs baseline: 3.7406x; 1.0129x over previous
import jax
import jax.numpy as jnp
from jax import lax
from jax.experimental import pallas as pl
from jax.experimental.pallas import tpu as pltpu
from jax.experimental.pallas import tpu_sc as plsc

B = 16384
EMBED = 64
HID = 256
NC, NS = 2, 16
NW = NC * NS
ROWS_PER_W = B // NW
BLK = 128
NBLK = ROWS_PER_W // BLK
LANES = 16
NCHUNK = HID // LANES


def _tc_project(g_ref, a_ref, m_ref, r_ref, w1_ref, b1_ref, b2_ref,
                pgr_o, pa_o, pm_o, b2v_o):
    w1 = w1_ref[...]
    dn = (((1,), (1,)), ((), ()))
    pg = lax.dot_general(g_ref[...], w1[:, 0:64], dn,
                         preferred_element_type=jnp.float32)
    pa = lax.dot_general(a_ref[...], w1[:, 64:128], dn,
                         preferred_element_type=jnp.float32)
    pm = lax.dot_general(m_ref[...], w1[:, 128:192], dn,
                         preferred_element_type=jnp.float32)
    pr = lax.dot_general(r_ref[...], w1[:, 192:256], dn,
                         preferred_element_type=jnp.float32)
    b1 = b1_ref[...]
    pgr = (pg[:, None, :] + pr[None, :, :]).reshape(16, HID) + b1
    pgr_o[...] = pgr
    pa_o[...] = pa
    pm_o[...] = pm
    lane = lax.broadcasted_iota(jnp.int32, (1, LANES), 1)
    b2v_o[...] = jnp.where(lane == 0, 1.0, 0.0) * b2_ref[...]


def _sc_forward(pgr_hbm, pa_hbm, pm_hbm, w2_hbm, b2v_hbm,
                g_hbm, a_hbm, r_hbm, m_hbm, out_hbm,
                pgr_v, pa_v, pm_buf, w2_v, b2_v,
                g_v, a_v, r_v, maj_v, out_v, part_v, offgr_v,
                sem0, sem1):
    wid = lax.axis_index("s") * NC + lax.axis_index("c")
    base = wid * ROWS_PER_W
    bsl = pl.ds(base, ROWS_PER_W)

    pltpu.sync_copy(pgr_hbm, pgr_v)
    pltpu.sync_copy(pa_hbm, pa_v)
    pltpu.sync_copy(w2_hbm, w2_v)
    pltpu.sync_copy(b2v_hbm, b2_v)
    pltpu.sync_copy(g_hbm.at[bsl], g_v.at[pl.ds(0, ROWS_PER_W)])
    pltpu.sync_copy(a_hbm.at[bsl], a_v.at[pl.ds(0, ROWS_PER_W)])
    pltpu.sync_copy(r_hbm.at[bsl], r_v.at[pl.ds(0, ROWS_PER_W)])
    pltpu.sync_copy(m_hbm.at[bsl], maj_v)

    sems = [sem0, sem1]
    descs = [None, None]
    descs[0] = pltpu.async_copy(pm_hbm.at[maj_v.at[pl.ds(0, BLK)]],
                                pm_buf.at[0], sems[0])

    @pl.loop(0, ROWS_PER_W // LANES)
    def _pre(i):
        s = pl.ds(i * LANES, LANES)
        offgr_v[s] = g_v[s] * 8 + r_v[s]

    w2c = [w2_v[0, pl.ds(LANES * j, LANES)] for j in range(NCHUNK)]
    b2acc = b2_v[0, pl.ds(0, LANES)]
    zero16 = b2acc * 0.0

    for blk in range(NBLK):
        cur = blk % 2
        if blk + 1 < NBLK:
            nxt = (blk + 1) % 2
            descs[nxt] = pltpu.async_copy(
                pm_hbm.at[maj_v.at[pl.ds((blk + 1) * BLK, BLK)]],
                pm_buf.at[nxt], sems[nxt])
        descs[cur].wait()
        rows = pm_buf.at[cur]

        @pl.loop(0, BLK, unroll=2)
        def _row(r):
            rb = blk * BLK + r
            row_gr = offgr_v[pl.ds(rb, LANES)][0]
            row_a = a_v[pl.ds(rb, LANES)][0]
            accs = [b2acc, zero16, zero16, zero16]
            for j in range(NCHUNK):
                v = rows[r, pl.ds(LANES * j, LANES)]
                v = v + pa_v[row_a, pl.ds(LANES * j, LANES)]
                v = v + pgr_v[row_gr, pl.ds(LANES * j, LANES)]
                h = jnp.maximum(v, 0.0)
                accs[j % 4] = accs[j % 4] + h * w2c[j]
            part_v[pl.ds(r * LANES, LANES)] = ((accs[0] + accs[1])
                                               + (accs[2] + accs[3]))

        iota16 = lax.iota(jnp.int32, LANES) * LANES

        @pl.loop(0, BLK // LANES)
        def _red(g16):
            gbase = g16 * (LANES * LANES)
            tot = plsc.load_gather(part_v, [iota16 + gbase])
            for l in range(1, LANES):
                tot = tot + plsc.load_gather(part_v, [iota16 + (gbase + l)])
            out_v[pl.ds(blk * BLK + g16 * LANES, LANES)] = tot

    pltpu.sync_copy(out_v, out_hbm.at[bsl])


def kernel(gender, age, major, grade, gender_tab, age_tab, major_tab,
           grade_tab, W1, b1, W2, b2):
    pgr, pa, pm, b2v = pl.pallas_call(
        _tc_project,
        out_shape=[
            jax.ShapeDtypeStruct((16, HID), jnp.float32),
            jax.ShapeDtypeStruct((100, HID), jnp.float32),
            jax.ShapeDtypeStruct((1000, HID), jnp.float32),
            jax.ShapeDtypeStruct((1, LANES), jnp.float32),
        ],
    )(gender_tab, age_tab, major_tab, grade_tab, W1,
      b1.reshape(1, HID), b2.reshape(1, 1))

    mesh = plsc.VectorSubcoreMesh(core_axis_name="c", subcore_axis_name="s",
                                  num_cores=NC, num_subcores=NS)
    sc = pl.kernel(
        _sc_forward,
        out_type=jax.ShapeDtypeStruct((B,), jnp.float32),
        mesh=mesh,
        compiler_params=pltpu.CompilerParams(needs_layout_passes=False),
        scratch_types=[
            pltpu.VMEM((16, HID), jnp.float32),
            pltpu.VMEM((100, HID), jnp.float32),
            pltpu.VMEM((2, BLK, HID), jnp.float32),
            pltpu.VMEM((1, HID), jnp.float32),
            pltpu.VMEM((1, LANES), jnp.float32),
            pltpu.VMEM((ROWS_PER_W + LANES,), jnp.int32),
            pltpu.VMEM((ROWS_PER_W + LANES,), jnp.int32),
            pltpu.VMEM((ROWS_PER_W + LANES,), jnp.int32),
            pltpu.VMEM((ROWS_PER_W,), jnp.int32),
            pltpu.VMEM((ROWS_PER_W,), jnp.float32),
            pltpu.VMEM((BLK * LANES,), jnp.float32),
            pltpu.VMEM((ROWS_PER_W + LANES,), jnp.int32),
            pltpu.SemaphoreType.DMA,
            pltpu.SemaphoreType.DMA,
        ],
    )
    i32 = jnp.int32
    return sc(pgr, pa, pm, W2, b2v,
              gender.astype(i32), age.astype(i32), grade.astype(i32),
              major.astype(i32))


# speedup vs baseline: 3.8557x; 1.0308x over previous
import jax
import jax.numpy as jnp
from jax import lax
from jax.experimental import pallas as pl
from jax.experimental.pallas import tpu as pltpu
from jax.experimental.pallas import tpu_sc as plsc

B = 16384
EMBED = 64
HID = 256
NC, NS = 2, 16
NW = NC * NS
ROWS_PER_W = B // NW
BLK = 128
NBLK = ROWS_PER_W // BLK
LANES = 16
NCHUNK = HID // LANES
NCHUNK2 = HID // (2 * LANES)


def _tc_project(g_ref, a_ref, m_ref, r_ref, w1_ref, b1_ref, w2_ref, b2_ref,
                pgr_o, pa_o, pm_o, sgn_o, b2v_o):
    w1 = w1_ref[...]
    dn = (((1,), (1,)), ((), ()))
    pg = lax.dot_general(g_ref[...], w1[:, 0:64], dn,
                         preferred_element_type=jnp.float32)
    pa = lax.dot_general(a_ref[...], w1[:, 64:128], dn,
                         preferred_element_type=jnp.float32)
    pm = lax.dot_general(m_ref[...], w1[:, 128:192], dn,
                         preferred_element_type=jnp.float32)
    pr = lax.dot_general(r_ref[...], w1[:, 192:256], dn,
                         preferred_element_type=jnp.float32)
    b1 = b1_ref[...]
    pgr = (pg[:, None, :] + pr[None, :, :]).reshape(16, HID) + b1
    w2 = w2_ref[...]
    aw = jnp.abs(w2)
    pgr_o[...] = (pgr * aw).astype(jnp.bfloat16)
    pa_o[...] = (pa * aw).astype(jnp.bfloat16)
    pm_o[...] = (pm * aw).astype(jnp.bfloat16)
    sgn_o[...] = jnp.sign(w2).astype(jnp.bfloat16)
    lane = lax.broadcasted_iota(jnp.int32, (1, LANES), 1)
    b2v_o[...] = jnp.where(lane == 0, 1.0, 0.0) * b2_ref[...]


def _sc_forward(pgr_hbm, pa_hbm, pm_hbm, sgn_hbm, b2v_hbm,
                g_hbm, a_hbm, r_hbm, m_hbm, out_hbm,
                pgr_v, pa_v, pm_buf, sgn_v, b2_v,
                g_v, a_v, r_v, maj_v, out_v, part_v, offgr_v,
                sem0, sem1):
    wid = lax.axis_index("s") * NC + lax.axis_index("c")
    base = wid * ROWS_PER_W
    bsl = pl.ds(base, ROWS_PER_W)

    pltpu.sync_copy(pgr_hbm, pgr_v)
    pltpu.sync_copy(pa_hbm, pa_v)
    pltpu.sync_copy(sgn_hbm, sgn_v)
    pltpu.sync_copy(b2v_hbm, b2_v)
    pltpu.sync_copy(g_hbm.at[bsl], g_v.at[pl.ds(0, ROWS_PER_W)])
    pltpu.sync_copy(a_hbm.at[bsl], a_v.at[pl.ds(0, ROWS_PER_W)])
    pltpu.sync_copy(r_hbm.at[bsl], r_v.at[pl.ds(0, ROWS_PER_W)])
    pltpu.sync_copy(m_hbm.at[bsl], maj_v)

    sems = [sem0, sem1]
    descs = [None, None]
    descs[0] = pltpu.async_copy(pm_hbm.at[maj_v.at[pl.ds(0, BLK)]],
                                pm_buf.at[0], sems[0])

    @pl.loop(0, ROWS_PER_W // LANES)
    def _pre(i):
        s = pl.ds(i * LANES, LANES)
        offgr_v[s] = g_v[s] * 8 + r_v[s]

    sgnc = [plsc.bitcast(sgn_v[0, pl.ds(LANES * j, LANES)], jnp.bfloat16)
            for j in range(NCHUNK2)]
    b2acc = b2_v[0, pl.ds(0, LANES)]
    zero16 = b2acc * 0.0
    zbf = jnp.zeros((2 * LANES,), jnp.bfloat16)

    for blk in range(NBLK):
        cur = blk % 2
        if blk + 1 < NBLK:
            nxt = (blk + 1) % 2
            descs[nxt] = pltpu.async_copy(
                pm_hbm.at[maj_v.at[pl.ds((blk + 1) * BLK, BLK)]],
                pm_buf.at[nxt], sems[nxt])
        descs[cur].wait()
        rows = pm_buf.at[cur]

        @pl.loop(0, BLK, unroll=2)
        def _row(r):
            rb = blk * BLK + r
            row_gr = offgr_v[pl.ds(rb, LANES)][0]
            row_a = a_v[pl.ds(rb, LANES)][0]
            acc_a = [b2acc, zero16]
            acc_b = [zero16, zero16]
            for j in range(NCHUNK2):
                s = pl.ds(LANES * j, LANES)
                vm = plsc.bitcast(rows[r, s], jnp.bfloat16)
                va = plsc.bitcast(pa_v[row_a, s], jnp.bfloat16)
                vg = plsc.bitcast(pgr_v[row_gr, s], jnp.bfloat16)
                v = (vm + va) + vg
                t = jnp.maximum(v, zbf) * sgnc[j]
                ta, tb = plsc.unpack(t, format=plsc.PackFormat.INTERLEAVED,
                                     preferred_element_type=jnp.float32)
                acc_a[j % 2] = acc_a[j % 2] + ta
                acc_b[j % 2] = acc_b[j % 2] + tb
            part_v[pl.ds(r * LANES, LANES)] = ((acc_a[0] + acc_a[1])
                                               + (acc_b[0] + acc_b[1]))

        iota16 = lax.iota(jnp.int32, LANES) * LANES

        @pl.loop(0, BLK // LANES)
        def _red(g16):
            gbase = g16 * (LANES * LANES)
            tot = plsc.load_gather(part_v, [iota16 + gbase])
            for l in range(1, LANES):
                tot = tot + plsc.load_gather(part_v, [iota16 + (gbase + l)])
            out_v[pl.ds(blk * BLK + g16 * LANES, LANES)] = tot

    pltpu.sync_copy(out_v, out_hbm.at[bsl])


def kernel(gender, age, major, grade, gender_tab, age_tab, major_tab,
           grade_tab, W1, b1, W2, b2):
    pgr, pa, pm, sgn, b2v = pl.pallas_call(
        _tc_project,
        out_shape=[
            jax.ShapeDtypeStruct((16, HID), jnp.bfloat16),
            jax.ShapeDtypeStruct((100, HID), jnp.bfloat16),
            jax.ShapeDtypeStruct((1000, HID), jnp.bfloat16),
            jax.ShapeDtypeStruct((1, HID), jnp.bfloat16),
            jax.ShapeDtypeStruct((1, LANES), jnp.float32),
        ],
    )(gender_tab, age_tab, major_tab, grade_tab, W1,
      b1.reshape(1, HID), W2, b2.reshape(1, 1))

    mesh = plsc.VectorSubcoreMesh(core_axis_name="c", subcore_axis_name="s",
                                  num_cores=NC, num_subcores=NS)
    sc = pl.kernel(
        _sc_forward,
        out_type=jax.ShapeDtypeStruct((B,), jnp.float32),
        mesh=mesh,
        compiler_params=pltpu.CompilerParams(needs_layout_passes=False),
        scratch_types=[
            pltpu.VMEM((16, HID // 2), jnp.int32),
            pltpu.VMEM((100, HID // 2), jnp.int32),
            pltpu.VMEM((2, BLK, HID // 2), jnp.int32),
            pltpu.VMEM((1, HID // 2), jnp.int32),
            pltpu.VMEM((1, LANES), jnp.float32),
            pltpu.VMEM((ROWS_PER_W + LANES,), jnp.int32),
            pltpu.VMEM((ROWS_PER_W + LANES,), jnp.int32),
            pltpu.VMEM((ROWS_PER_W + LANES,), jnp.int32),
            pltpu.VMEM((ROWS_PER_W,), jnp.int32),
            pltpu.VMEM((ROWS_PER_W,), jnp.float32),
            pltpu.VMEM((BLK * LANES,), jnp.float32),
            pltpu.VMEM((ROWS_PER_W + LANES,), jnp.int32),
            pltpu.SemaphoreType.DMA,
            pltpu.SemaphoreType.DMA,
        ],
    )
    i32 = jnp.int32
    h2 = HID // 2
    pm = lax.bitcast_convert_type(pm.reshape(1000, h2, 2), i32)
    pgr = lax.bitcast_convert_type(pgr.reshape(16, h2, 2), i32)
    pa = lax.bitcast_convert_type(pa.reshape(100, h2, 2), i32)
    sgn = lax.bitcast_convert_type(sgn.reshape(1, h2, 2), i32)
    return sc(pgr, pa, pm, sgn, b2v,
              gender.astype(i32), age.astype(i32), grade.astype(i32),
              major.astype(i32))
